# Initial kernel scaffold; baseline (speedup 1.0000x reference)
import jax
import jax.numpy as jnp
from jax.experimental import pallas as pl


def kernel(embeddings, originalIndexes):
    raise NotImplementedError("write your pallas kernel here")


# speedup vs baseline: 20.3672x; 20.3672x over previous
import functools

import jax
import jax.numpy as jnp
from jax import lax
from jax.experimental import pallas as pl
from jax.experimental.pallas import tpu as pltpu
from jax.experimental.pallas import tpu_sc as plsc

N = 3072
D = 1024
NA = N // 3
RB = 384
AB = RB // 3
GRID = N // RB


@functools.lru_cache(maxsize=None)
def _make_sc_gather():
    info = plsc.get_sparse_core_info()
    nw = info.num_cores * info.num_subcores
    rows_per_w = N // nw

    mesh = plsc.VectorSubcoreMesh(core_axis_name="c", subcore_axis_name="s")

    @functools.partial(
        pl.kernel,
        mesh=mesh,
        out_type=jax.ShapeDtypeStruct((N, D), jnp.float32),
        scratch_types=[
            pltpu.VMEM((rows_per_w,), jnp.int32),
            pltpu.VMEM((rows_per_w, D), jnp.float32),
            pltpu.SemaphoreType.DMA,
        ],
    )
    def gather_k(table_hbm, idx_hbm, out_hbm, idx_v, rows_v, sem):
        wid = lax.axis_index("s") * info.num_cores + lax.axis_index("c")
        base = wid * rows_per_w
        pltpu.sync_copy(idx_hbm.at[pl.ds(base, rows_per_w)], idx_v)
        pltpu.async_copy(table_hbm.at[idx_v], rows_v, sem).wait()
        pltpu.sync_copy(rows_v, out_hbm.at[pl.ds(base, rows_per_w)])

    return gather_k


def _normalize_body(e_ref, out_ref):
    e = e_ref[...]
    nrm = jnp.sqrt(jnp.sum(e * e, axis=1, keepdims=True))
    out_ref[...] = e / jnp.maximum(nrm, 1e-12)


def _dist_body(a_ref, b_ref, dist_ref, ranks_ref, medr_ref):
    i = pl.program_id(0)
    a = a_ref[...]
    b = b_ref[...]
    d = 1.0 - lax.dot_general(
        a, b, (((1,), (1,)), ((), ())), preferred_element_type=jnp.float32
    )

    r0 = i * RB
    row = r0 + lax.broadcasted_iota(jnp.int32, (RB, 1), 0)
    col = lax.broadcasted_iota(jnp.int32, (RB, N), 1)
    is_anchor = (row % 3) == 0
    d = jnp.where(is_anchor & (col == row), -1.0, d)
    dist_ref[...] = d

    tcol = row + 1
    v = jnp.sum(jnp.where(col == tcol, d, 0.0), axis=1, keepdims=True)
    less = jnp.sum((d < v).astype(jnp.float32), axis=1, keepdims=True)
    eqb = jnp.sum(((d == v) & (col < tcol)).astype(jnp.float32),
                  axis=1, keepdims=True)
    rank_all = less + eqb - 1.0

    ja = lax.broadcasted_iota(jnp.int32, (AB, RB), 0)
    ka = lax.broadcasted_iota(jnp.int32, (AB, RB), 1)
    sel = (ka == 3 * ja).astype(jnp.float32)
    ranks = lax.dot_general(
        sel, rank_all, (((1,), (0,)), ((), ())),
        preferred_element_type=jnp.float32,
    )
    ranks_ref[...] = ranks.astype(jnp.int32)

    part = jnp.sum(ranks)

    @pl.when(i == 0)
    def _():
        medr_ref[0, 0] = part

    @pl.when(i > 0)
    def _():
        medr_ref[0, 0] = medr_ref[0, 0] + part

    @pl.when(i == GRID - 1)
    def _():
        medr_ref[0, 0] = medr_ref[0, 0] / float(NA)


def kernel(embeddings, originalIndexes):
    e = _make_sc_gather()(embeddings, originalIndexes)

    normed = pl.pallas_call(
        _normalize_body,
        grid=(GRID,),
        in_specs=[pl.BlockSpec((RB, D), lambda i: (i, 0))],
        out_specs=pl.BlockSpec((RB, D), lambda i: (i, 0)),
        out_shape=jax.ShapeDtypeStruct((N, D), jnp.float32),
    )(e)

    dist, ranks2d, medr = pl.pallas_call(
        _dist_body,
        grid=(GRID,),
        in_specs=[
            pl.BlockSpec((RB, D), lambda i: (i, 0)),
            pl.BlockSpec((N, D), lambda i: (0, 0)),
        ],
        out_specs=[
            pl.BlockSpec((RB, N), lambda i: (i, 0)),
            pl.BlockSpec((AB, 1), lambda i: (i, 0)),
            pl.BlockSpec(memory_space=pltpu.SMEM),
        ],
        out_shape=[
            jax.ShapeDtypeStruct((N, N), jnp.float32),
            jax.ShapeDtypeStruct((NA, 1), jnp.int32),
            jax.ShapeDtypeStruct((1, 1), jnp.float32),
        ],
    )(normed, normed)

    return dist, ranks2d.reshape(NA), medr[0, 0]


# speedup vs baseline: 21.1286x; 1.0374x over previous
import functools

import jax
import jax.numpy as jnp
from jax import lax
from jax.experimental import pallas as pl
from jax.experimental.pallas import tpu as pltpu
from jax.experimental.pallas import tpu_sc as plsc

N = 3072
D = 1024
NA = N // 3
RB = 384
AB = RB // 3
GRID = N // RB


@functools.lru_cache(maxsize=None)
def _make_sc_gather():
    info = plsc.get_sparse_core_info()
    nw = info.num_cores * info.num_subcores
    rows_per_w = N // nw

    mesh = plsc.VectorSubcoreMesh(core_axis_name="c", subcore_axis_name="s")

    @functools.partial(
        pl.kernel,
        mesh=mesh,
        out_type=jax.ShapeDtypeStruct((N, D), jnp.float32),
        scratch_types=[
            pltpu.VMEM((rows_per_w,), jnp.int32),
            pltpu.VMEM((rows_per_w, D), jnp.float32),
            pltpu.SemaphoreType.DMA,
        ],
    )
    def gather_k(table_hbm, idx_hbm, out_hbm, idx_v, rows_v, sem):
        wid = lax.axis_index("s") * info.num_cores + lax.axis_index("c")
        base = wid * rows_per_w
        pltpu.sync_copy(idx_hbm.at[pl.ds(base, rows_per_w)], idx_v)
        pltpu.async_copy(table_hbm.at[idx_v], rows_v, sem).wait()
        pltpu.sync_copy(rows_v, out_hbm.at[pl.ds(base, rows_per_w)])

    return gather_k


def _normalize_body(e_ref, out_ref):
    e = e_ref[...]
    nrm = jnp.sqrt(jnp.sum(e * e, axis=1, keepdims=True))
    out_ref[...] = (e / jnp.maximum(nrm, 1e-12)).astype(jnp.bfloat16)


def _dist_body(a_ref, b_ref, dist_ref, ranks_ref, medr_ref):
    i = pl.program_id(0)
    a = a_ref[...]
    b = b_ref[...]
    d = 1.0 - lax.dot_general(
        a, b, (((1,), (1,)), ((), ())), preferred_element_type=jnp.float32
    )

    r0 = i * RB
    row = r0 + lax.broadcasted_iota(jnp.int32, (RB, 1), 0)
    col = lax.broadcasted_iota(jnp.int32, (RB, N), 1)
    is_anchor = (row % 3) == 0
    d = jnp.where(is_anchor & (col == row), -1.0, d)
    dist_ref[...] = d

    tcol = row + 1
    v = jnp.sum(jnp.where(col == tcol, d, 0.0), axis=1, keepdims=True)
    less = jnp.sum((d < v).astype(jnp.float32), axis=1, keepdims=True)
    eqb = jnp.sum(((d == v) & (col < tcol)).astype(jnp.float32),
                  axis=1, keepdims=True)
    rank_all = less + eqb - 1.0

    ja = lax.broadcasted_iota(jnp.int32, (AB, RB), 0)
    ka = lax.broadcasted_iota(jnp.int32, (AB, RB), 1)
    sel = (ka == 3 * ja).astype(jnp.float32)
    ranks = lax.dot_general(
        sel, rank_all, (((1,), (0,)), ((), ())),
        preferred_element_type=jnp.float32,
    )
    ranks_ref[...] = ranks.astype(jnp.int32)

    part = jnp.sum(ranks)

    @pl.when(i == 0)
    def _():
        medr_ref[0, 0] = part

    @pl.when(i > 0)
    def _():
        medr_ref[0, 0] = medr_ref[0, 0] + part

    @pl.when(i == GRID - 1)
    def _():
        medr_ref[0, 0] = medr_ref[0, 0] / float(NA)


def kernel(embeddings, originalIndexes):
    e = _make_sc_gather()(embeddings, originalIndexes)

    normed = pl.pallas_call(
        _normalize_body,
        grid=(GRID,),
        in_specs=[pl.BlockSpec((RB, D), lambda i: (i, 0))],
        out_specs=pl.BlockSpec((RB, D), lambda i: (i, 0)),
        out_shape=jax.ShapeDtypeStruct((N, D), jnp.bfloat16),
    )(e)

    dist, ranks2d, medr = pl.pallas_call(
        _dist_body,
        grid=(GRID,),
        in_specs=[
            pl.BlockSpec((RB, D), lambda i: (i, 0)),
            pl.BlockSpec((N, D), lambda i: (0, 0)),
        ],
        out_specs=[
            pl.BlockSpec((RB, N), lambda i: (i, 0)),
            pl.BlockSpec((AB, 1), lambda i: (i, 0)),
            pl.BlockSpec(memory_space=pltpu.SMEM),
        ],
        out_shape=[
            jax.ShapeDtypeStruct((N, N), jnp.float32),
            jax.ShapeDtypeStruct((NA, 1), jnp.int32),
            jax.ShapeDtypeStruct((1, 1), jnp.float32),
        ],
    )(normed, normed)

    return dist, ranks2d.reshape(NA), medr[0, 0]
